# baseline (speedup 1.0000x reference)
import functools

import jax
import jax.numpy as jnp
from jax import lax
from jax.experimental import pallas as pl
from jax.experimental.pallas import tpu as pltpu
from jax.experimental.pallas import tpu_sc as plsc

NC = 2
NS = 16
L = 16
B = 8
N = 2048
M = 2048
WPB = 4
CHUNK = N // WPB

_mesh = plsc.VectorSubcoreMesh(core_axis_name="c", subcore_axis_name="s", num_cores=NC, num_subcores=NS)


@functools.partial(
    pl.kernel,
    out_type=(
        jax.ShapeDtypeStruct((B, N), jnp.float32),
        jax.ShapeDtypeStruct((B, M), jnp.float32),
        jax.ShapeDtypeStruct((B, N), jnp.int32),
        jax.ShapeDtypeStruct((B, M), jnp.int32),
    ),
    mesh=_mesh,
    compiler_params=pltpu.CompilerParams(needs_layout_passes=False),
    scratch_types=dict(
        x1v=pltpu.VMEM((CHUNK * 3 + L,), jnp.float32),
        x1r=pltpu.VMEM((CHUNK * 3 + L,), jnp.float32),
        x2v=pltpu.VMEM((3 * M,), jnp.float32),
        s2v=pltpu.VMEM((M,), jnp.float32),
        rminv=pltpu.VMEM((CHUNK,), jnp.float32),
        ridxv=pltpu.VMEM((CHUNK,), jnp.int32),
        cminv=pltpu.VMEM((M,), jnp.float32),
        cidxv=pltpu.VMEM((M,), jnp.int32),
        mmin=pltpu.VMEM((WPB * M,), jnp.float32),
        midx=pltpu.VMEM((WPB * M,), jnp.int32),
        shmin=pltpu.VMEM_SHARED((NS * M,), jnp.float32),
        shidx=pltpu.VMEM_SHARED((NS * M,), jnp.int32),
    ),
)
def _chamfer_sc(x1_hbm, x2_hbm, d1_hbm, d2_hbm, i1_hbm, i2_hbm,
                x1v, x1r, x2v, s2v, rminv, ridxv, cminv, cidxv,
                mmin, midx, shmin, shidx):
  c = lax.axis_index("c")
  s = lax.axis_index("s")
  wid = c * NS + s
  b = wid // WPB
  chunk = wid % WPB
  row0 = chunk * CHUNK

  pltpu.sync_copy(x1_hbm.at[b, pl.ds(row0 * 3, CHUNK * 3)],
                  x1v.at[pl.ds(0, CHUNK * 3)])
  pltpu.sync_copy(x2_hbm.at[b], x2v)

  lanes = lax.iota(jnp.int32, L)
  inf16 = jnp.full((L,), jnp.inf, jnp.float32)
  zero16 = jnp.zeros((L,), jnp.int32)

  def _bf16r(v):
    u = plsc.bitcast(v, jnp.uint32)
    u = (u + jnp.uint32(0x7FFF) + ((u >> jnp.uint32(16)) & jnp.uint32(1)))
    u = u & jnp.uint32(0xFFFF0000)
    return plsc.bitcast(u, jnp.float32)

  def _prep(jc, _):
    off = jc * L
    b0 = x2v[pl.ds(off, L)]
    b1 = x2v[pl.ds(M + off, L)]
    b2 = x2v[pl.ds(2 * M + off, L)]
    s2v[pl.ds(off, L)] = (b0 * b0 + b1 * b1) + b2 * b2
    x2v[pl.ds(off, L)] = _bf16r(b0)
    x2v[pl.ds(M + off, L)] = _bf16r(b1)
    x2v[pl.ds(2 * M + off, L)] = _bf16r(b2)
    cminv[pl.ds(off, L)] = inf16
    cidxv[pl.ds(off, L)] = zero16
    return 0
  lax.fori_loop(0, M // L, _prep, 0)

  def _prep1(jc, _):
    off = jc * L
    x1r[pl.ds(off, L)] = _bf16r(x1v[pl.ds(off, L)])
    return 0
  lax.fori_loop(0, (CHUNK * 3 + L) // L, _prep1, 0)

  def _row(r, carry):
    accm, acci, g = carry
    i = g * L + r
    va = x1v[pl.ds(3 * i, L)]
    A0 = jnp.full((L,), va[0], jnp.float32)
    A1 = jnp.full((L,), va[1], jnp.float32)
    A2 = jnp.full((L,), va[2], jnp.float32)
    s1r = (A0 * A0 + A1 * A1) + A2 * A2
    vb = x1r[pl.ds(3 * i, L)]
    a0 = jnp.full((L,), vb[0], jnp.float32)
    a1 = jnp.full((L,), vb[1], jnp.float32)
    a2 = jnp.full((L,), vb[2], jnp.float32)
    iv = jnp.full((L,), row0 + i, jnp.int32)

    def _col(jc, carry):
      rmin, ridx = carry
      off = jc * L
      b0 = x2v[pl.ds(off, L)]
      b1 = x2v[pl.ds(M + off, L)]
      b2 = x2v[pl.ds(2 * M + off, L)]
      s2c = s2v[pl.ds(off, L)]
      cross = (a0 * b0 + a1 * b1) + a2 * b2
      d = (s1r + s2c) - 2.0 * cross
      jv = lanes + off
      mr = d < rmin
      rmin = jnp.where(mr, d, rmin)
      ridx = jnp.where(mr, jv, ridx)
      cmin = cminv[pl.ds(off, L)]
      cidx = cidxv[pl.ds(off, L)]
      mc = d < cmin
      cminv[pl.ds(off, L)] = jnp.where(mc, d, cmin)
      cidxv[pl.ds(off, L)] = jnp.where(mc, iv, cidx)
      return rmin, ridx

    rmin, ridx = lax.fori_loop(0, M // L, _col, (inf16, zero16))
    rs = jnp.min(rmin)
    ri = jnp.min(jnp.where(rmin == rs, ridx, jnp.int32(M)))
    lm = lanes == r
    accm = jnp.where(lm, rs, accm)
    acci = jnp.where(lm, ri, acci)
    return accm, acci, g

  def _rowgrp(g, _):
    accm, acci, _g = lax.fori_loop(0, L, _row, (inf16, zero16, g))
    rminv[pl.ds(g * L, L)] = accm
    ridxv[pl.ds(g * L, L)] = acci
    return 0
  lax.fori_loop(0, CHUNK // L, _rowgrp, 0)

  pltpu.sync_copy(rminv, d1_hbm.at[b, pl.ds(row0, CHUNK)])
  pltpu.sync_copy(ridxv, i1_hbm.at[b, pl.ds(row0, CHUNK)])

  pltpu.sync_copy(cminv, shmin.at[pl.ds(s * M, M)])
  pltpu.sync_copy(cidxv, shidx.at[pl.ds(s * M, M)])
  plsc.subcore_barrier()

  @pl.when(chunk == 0)
  def _merge():
    pltpu.sync_copy(shmin.at[pl.ds(s * M, WPB * M)], mmin)
    pltpu.sync_copy(shidx.at[pl.ds(s * M, WPB * M)], midx)

    def _mrg(jc, _):
      off = jc * L
      m = mmin[pl.ds(off, L)]
      ix = midx[pl.ds(off, L)]
      for k in range(1, WPB):
        mk = mmin[pl.ds(k * M + off, L)]
        ik = midx[pl.ds(k * M + off, L)]
        lt = mk < m
        m = jnp.where(lt, mk, m)
        ix = jnp.where(lt, ik, ix)
      cminv[pl.ds(off, L)] = m
      cidxv[pl.ds(off, L)] = ix
      return 0
    lax.fori_loop(0, M // L, _mrg, 0)
    pltpu.sync_copy(cminv, d2_hbm.at[b])
    pltpu.sync_copy(cidxv, i2_hbm.at[b])


@jax.jit
def kernel(input1, input2):
  x1f = input1.reshape(B, N * 3)
  x2t = jnp.swapaxes(input2, 1, 2).reshape(B, 3 * M)
  d1, d2, i1, i2 = _chamfer_sc(x1f, x2t)
  return d1, d2, i1, i2
